# baseline (speedup 1.0000x reference)
import functools

import jax
import jax.numpy as jnp
from jax import lax
from jax.experimental import pallas as pl
from jax.experimental.pallas import tpu as pltpu
from jax.experimental.pallas import tpu_sc as plsc

USER_NUM = 1000000
ITEM_NUM = 1000000
EMB_DIM = 16
BATCH = 16384

NUM_CORES = 2
NUM_SUBCORES = 16
NUM_WORKERS = NUM_CORES * NUM_SUBCORES
ROWS_PER_WORKER = BATCH // NUM_WORKERS


def _sc_gather_kernel(uemb_hbm, iemb_hbm, ibias_hbm, uidx_hbm, iidx_hbm,
                      uout_hbm, iout_hbm, bout_hbm,
                      uidx_v, iidx_v, urows_v, irows_v, bias_v,
                      sem_u, sem_i, sem_b):
    wid = lax.axis_index("s") * NUM_CORES + lax.axis_index("c")
    base = wid * ROWS_PER_WORKER
    pltpu.sync_copy(uidx_hbm.at[pl.ds(base, ROWS_PER_WORKER)], uidx_v)
    pltpu.sync_copy(iidx_hbm.at[pl.ds(base, ROWS_PER_WORKER)], iidx_v)
    cu = pltpu.async_copy(uemb_hbm.at[uidx_v], urows_v, sem_u)
    ci = pltpu.async_copy(iemb_hbm.at[iidx_v], irows_v, sem_i)
    cb = pltpu.async_copy(ibias_hbm.at[iidx_v], bias_v, sem_b)
    cu.wait()
    ci.wait()
    cb.wait()
    pltpu.sync_copy(urows_v, uout_hbm.at[pl.ds(base, ROWS_PER_WORKER)])
    pltpu.sync_copy(irows_v, iout_hbm.at[pl.ds(base, ROWS_PER_WORKER)])
    pltpu.sync_copy(bias_v, bout_hbm.at[pl.ds(base, ROWS_PER_WORKER)])


def _sc_gather(user_embeddings, item_embeddings, item_bias, user, item):
    mesh = plsc.VectorSubcoreMesh(core_axis_name="c", subcore_axis_name="s")
    k = pl.kernel(
        _sc_gather_kernel,
        compiler_params=pltpu.CompilerParams(use_tc_tiling_on_sc=False),
        out_type=[
            jax.ShapeDtypeStruct((BATCH, EMB_DIM), jnp.float32),
            jax.ShapeDtypeStruct((BATCH, EMB_DIM), jnp.float32),
            jax.ShapeDtypeStruct((BATCH,), jnp.float32),
        ],
        mesh=mesh,
        scratch_types=[
            pltpu.VMEM((ROWS_PER_WORKER,), jnp.int32),
            pltpu.VMEM((ROWS_PER_WORKER,), jnp.int32),
            pltpu.VMEM((ROWS_PER_WORKER, EMB_DIM), jnp.float32),
            pltpu.VMEM((ROWS_PER_WORKER, EMB_DIM), jnp.float32),
            pltpu.VMEM((ROWS_PER_WORKER,), jnp.float32),
            pltpu.SemaphoreType.DMA,
            pltpu.SemaphoreType.DMA,
            pltpu.SemaphoreType.DMA,
        ],
    )
    return k(user_embeddings, item_embeddings, item_bias, user, item)


def _tc_bce_kernel(u_ref, i_ref, b_ref, t_ref, o_ref):
    prod = u_ref[...] * i_ref[...]
    logits = jnp.sum(prod, axis=0) + b_ref[...]
    t = t_ref[...]
    per = (jnp.maximum(logits, 0.0) - logits * t
           + jnp.log1p(jnp.exp(-jnp.abs(logits))))
    o_ref[...] = jnp.reshape(jnp.sum(per) * (1.0 / BATCH), (1, 1))


@jax.jit
def kernel(user, item, label, user_embeddings, item_embeddings, item_bias):
    user = user.astype(jnp.int32)
    item = item.astype(jnp.int32)
    u_rows, i_rows, b_g = _sc_gather(user_embeddings, item_embeddings,
                                     item_bias, user, item)
    u3 = u_rows.T.reshape(EMB_DIM, 128, 128)
    i3 = i_rows.T.reshape(EMB_DIM, 128, 128)
    b2 = b_g.reshape(128, 128)
    t2 = label.reshape(128, 128)
    loss = pl.pallas_call(
        _tc_bce_kernel,
        out_shape=jax.ShapeDtypeStruct((1, 1), jnp.float32),
    )(u3, i3, b2, t2)
    return loss.reshape(())


# speedup vs baseline: 1.0162x; 1.0162x over previous
import jax
import jax.numpy as jnp
from jax import lax
from jax.experimental import pallas as pl
from jax.experimental.pallas import tpu as pltpu
from jax.experimental.pallas import tpu_sc as plsc

USER_NUM = 1000000
ITEM_NUM = 1000000
EMB_DIM = 16
BATCH = 16384

NUM_CORES = 2
NUM_SUBCORES = 16
NUM_WORKERS = NUM_CORES * NUM_SUBCORES
ROWS_PER_WORKER = BATCH // NUM_WORKERS

L = 16
PHYS_W = 128
ROWS_PER_PHYS = PHYS_W // EMB_DIM
CHUNK = 256
N_CHUNK = ROWS_PER_WORKER // CHUNK


def _sc_logits_kernel(ue2_hbm, ie2_hbm, ibias_hbm, uidx_hbm, iidx_hbm,
                      logits_hbm,
                      uhi_v, ihi_v, ulo_v, ilo_v, iraw_v, tmp_v,
                      bias_v, logits_v, ug_v, ig_v,
                      sem_u, sem_i, sem_b):
    wid = lax.axis_index("s") * NUM_CORES + lax.axis_index("c")
    base = wid * ROWS_PER_WORKER

    pltpu.sync_copy(uidx_hbm.at[pl.ds(base, ROWS_PER_WORKER)], tmp_v)
    pltpu.sync_copy(iidx_hbm.at[pl.ds(base, ROWS_PER_WORKER)], iraw_v)

    cb = pltpu.async_copy(ibias_hbm.at[iraw_v], bias_v, sem_b)

    @pl.loop(0, ROWS_PER_WORKER, step=L)
    def _(j):
        uv = tmp_v[pl.ds(j, L)]
        iv = iraw_v[pl.ds(j, L)]
        uhi_v[pl.ds(j, L)] = jnp.right_shift(uv, 3)
        ulo_v[pl.ds(j, L)] = jnp.bitwise_and(uv, 7) * EMB_DIM
        ihi_v[pl.ds(j, L)] = jnp.right_shift(iv, 3)
        ilo_v[pl.ds(j, L)] = jnp.bitwise_and(iv, 7) * EMB_DIM

    cb.wait()

    for c in range(N_CHUNK):
        cu = pltpu.async_copy(
            ue2_hbm.at[uhi_v.at[pl.ds(c * CHUNK, CHUNK)]], ug_v, sem_u)
        ci = pltpu.async_copy(
            ie2_hbm.at[ihi_v.at[pl.ds(c * CHUNK, CHUNK)]], ig_v, sem_i)
        cu.wait()
        ci.wait()

        @pl.loop(0, CHUNK, step=L)
        def _(g):
            jvec = jnp.arange(L, dtype=jnp.int32) + g
            uo = ulo_v[pl.ds(c * CHUNK + g, L)]
            io = ilo_v[pl.ds(c * CHUNK + g, L)]
            acc = bias_v[pl.ds(c * CHUNK + g, L)]
            for d in range(EMB_DIM):
                ut = plsc.load_gather(ug_v, [jvec, uo + d])
                it = plsc.load_gather(ig_v, [jvec, io + d])
                acc = acc + ut * it
            logits_v[pl.ds(c * CHUNK + g, L)] = acc

    pltpu.sync_copy(logits_v, logits_hbm.at[pl.ds(base, ROWS_PER_WORKER)])


def _sc_logits(ue2, ie2, item_bias, user, item):
    mesh = plsc.VectorSubcoreMesh(core_axis_name="c", subcore_axis_name="s")
    k = pl.kernel(
        _sc_logits_kernel,
        out_type=jax.ShapeDtypeStruct((BATCH,), jnp.float32),
        mesh=mesh,
        compiler_params=pltpu.CompilerParams(needs_layout_passes=False),
        scratch_types=[
            pltpu.VMEM((ROWS_PER_WORKER,), jnp.int32),
            pltpu.VMEM((ROWS_PER_WORKER,), jnp.int32),
            pltpu.VMEM((ROWS_PER_WORKER,), jnp.int32),
            pltpu.VMEM((ROWS_PER_WORKER,), jnp.int32),
            pltpu.VMEM((ROWS_PER_WORKER,), jnp.int32),
            pltpu.VMEM((ROWS_PER_WORKER,), jnp.int32),
            pltpu.VMEM((ROWS_PER_WORKER,), jnp.float32),
            pltpu.VMEM((ROWS_PER_WORKER,), jnp.float32),
            pltpu.VMEM((CHUNK, PHYS_W), jnp.float32),
            pltpu.VMEM((CHUNK, PHYS_W), jnp.float32),
            pltpu.SemaphoreType.DMA,
            pltpu.SemaphoreType.DMA,
            pltpu.SemaphoreType.DMA,
        ],
    )
    return k(ue2, ie2, item_bias, user, item)


def _tc_bce_kernel(x_ref, t_ref, o_ref):
    logits = x_ref[...]
    t = t_ref[...]
    per = (jnp.maximum(logits, 0.0) - logits * t
           + jnp.log1p(jnp.exp(-jnp.abs(logits))))
    o_ref[...] = jnp.reshape(jnp.sum(per) * (1.0 / BATCH), (1, 1))


@jax.jit
def kernel(user, item, label, user_embeddings, item_embeddings, item_bias):
    user = user.astype(jnp.int32)
    item = item.astype(jnp.int32)
    ue2 = user_embeddings.reshape(USER_NUM * EMB_DIM // PHYS_W, PHYS_W)
    ie2 = item_embeddings.reshape(ITEM_NUM * EMB_DIM // PHYS_W, PHYS_W)
    logits = _sc_logits(ue2, ie2, item_bias, user, item)
    loss = pl.pallas_call(
        _tc_bce_kernel,
        out_shape=jax.ShapeDtypeStruct((1, 1), jnp.float32),
    )(logits.reshape(128, 128), label.reshape(128, 128))
    return loss.reshape(())
